# baseline (speedup 1.0000x reference)
import functools
import math

import jax
import jax.numpy as jnp
import numpy as np
from jax import lax
from jax.experimental import pallas as pl
from jax.experimental.pallas import tpu as pltpu
from jax.experimental.pallas import tpu_sc as plsc

N = 1048576
WIDTH = 1024
TOTAL_WIDTH = 100000
N_FREQ = 5
NC0 = 4096
STRIDE = 128

_FREQS = np.repeat(
    1.0 / 1000.0 ** (2.0 * np.arange(1, N_FREQ + 1) / N_FREQ), 2
).astype(np.float32)
_SHIFTS = np.tile(np.array([0.0, np.pi / 2.0], dtype=np.float32), N_FREQ)

_BC = (np.arange(8, dtype=np.float32) * 128.0)[:, None]
_SVC = np.sin(_BC * _FREQS[None, :] + _SHIFTS[None, :]).astype(np.float32)

_SVC_FLAT = _SVC.reshape(1, 80)
_G = np.kron(np.eye(8, dtype=np.float32), np.ones((8, 1), np.float32))
_GT = np.ascontiguousarray(_G.T)


def _table_body(freqs_ref, shifts_ref, svcf_ref, g_ref, gt_ref,
                lpi_ref, b0_ref, b1row_ref, w0a_ref, b0a_ref, w0b_ref,
                w1apt_ref, w1abbd_ref, b1at_ref, w1bbd_ref, out_ref):
    c0 = lax.broadcasted_iota(jnp.int32, (NC0, 1), 0).astype(jnp.float32)
    emb = jnp.sin(c0 * freqs_ref[...] + shifts_ref[...])

    lpi = lpi_ref[0, 0]
    const_in = -jnp.log(1.0 + jnp.exp(-lpi)) - math.log(16.0)
    lpo = -jnp.log(1.0 + jnp.exp(lpi)) - math.log(float(TOTAL_WIDTH - WIDTH))

    dot = lambda x, y: jnp.dot(x, y, preferred_element_type=jnp.float32)

    h0 = jax.nn.sigmoid(dot(emb, w0a_ref[...]) + b0a_ref[...])
    h0 = dot(h0, w0b_ref[...]) + b0_ref[...]
    m0 = jnp.max(h0, axis=1, keepdims=True)
    ls0 = h0 - m0 - jnp.log(jnp.sum(jnp.exp(h0 - m0), axis=1, keepdims=True))

    bbin_flat = dot(svcf_ref[...], w1abbd_ref[...]) + b1at_ref[...]
    s = jax.nn.sigmoid(dot(emb, w1apt_ref[...]) + bbin_flat)
    hall = dot(s, w1bbd_ref[...]) + b1row_ref[...]
    m1 = jnp.max(hall, axis=1, keepdims=True)
    l1 = m1 + jnp.log(dot(jnp.exp(hall - m1), g_ref[...]))

    out_ref[:, 0:64] = hall + dot(ls0 - l1, gt_ref[...]) + const_in
    out_ref[:, 64:128] = jnp.broadcast_to(lpo, (NC0, 64))


def _build_table(lpi, b0, b1, w0a, b0a, w0b, w1ap, w1ab, b1a, w1b):
    eye8 = jnp.eye(8, dtype=jnp.float32)
    w1apt = jnp.concatenate([w1ap] * 8, axis=1)
    w1abbd = jnp.kron(eye8, w1ab)
    w1bbd = jnp.kron(eye8, w1b)
    b1at = jnp.concatenate([b1a] * 8, axis=1)
    return pl.pallas_call(
        _table_body,
        out_shape=jax.ShapeDtypeStruct((NC0, STRIDE), jnp.float32),
    )(jnp.asarray(_FREQS[None, :]), jnp.asarray(_SHIFTS[None, :]),
      jnp.asarray(_SVC_FLAT), jnp.asarray(_G), jnp.asarray(_GT),
      lpi, b0, b1.reshape(1, 64), w0a, b0a, w0b, w1apt, w1abbd, b1at, w1bbd)


_NCORES = 2
_NSUB = 16
_NW = _NCORES * _NSUB
_PER_W = N // _NW
_CH = 4096
_NCH = _PER_W // _CH

_sc_mesh = plsc.VectorSubcoreMesh(core_axis_name="c", subcore_axis_name="s")


@functools.partial(
    pl.kernel,
    mesh=_sc_mesh,
    out_type=jax.ShapeDtypeStruct((N,), jnp.float32),
    scratch_types=[
        pltpu.VMEM((_CH,), jnp.int32), pltpu.VMEM((_CH,), jnp.int32),
        pltpu.VMEM((_CH,), jnp.int32), pltpu.VMEM((_CH,), jnp.int32),
        pltpu.VMEM((_CH,), jnp.float32), pltpu.VMEM((_CH,), jnp.float32),
        pltpu.SemaphoreType.DMA, pltpu.SemaphoreType.DMA,
        pltpu.SemaphoreType.DMA, pltpu.SemaphoreType.DMA,
        pltpu.SemaphoreType.DMA, pltpu.SemaphoreType.DMA,
        pltpu.VMEM_SHARED((NC0 * STRIDE,), jnp.float32),
    ],
)
def _sc_lookup(coords_hbm, table_hbm, out_hbm,
               cv0, cv1, ix0, ix1, y0, y1,
               si0, si1, sg0, sg1, so0, so1, sh_table):
    cv = (cv0, cv1)
    ixv = (ix0, ix1)
    yv = (y0, y1)
    sin_ = (si0, si1)
    sg = (sg0, sg1)
    so = (so0, so1)
    sid = lax.axis_index("s")
    wid = sid * _NCORES + lax.axis_index("c")
    base = wid * _PER_W

    @pl.when(sid == 0)
    def _stage_table():
        pltpu.sync_copy(table_hbm, sh_table)

    plsc.subcore_barrier()

    def start_in(ci, b):
        return pltpu.async_copy(
            coords_hbm.at[pl.ds(base + ci * _CH, _CH)], cv[b], sin_[b])

    def compute(b):
        def vec(vi, carry):
            v = cv[b][pl.ds(vi * 16, 16)]
            a = v & 0xFFFF
            fs = jnp.abs(lax.shift_right_logical(v, 16) - a)
            ixv[b][pl.ds(vi * 16, 16)] = (a << 7) + jnp.minimum(fs >> 4, 64)
            return carry
        lax.fori_loop(0, _CH // 16, vec, 0)

    h_in = start_in(0, 0)
    h_in.wait()
    compute(0)
    h_in = start_in(1, 1)
    h_out = [None, None]
    for ci in range(_NCH):
        b = ci & 1
        if h_out[b] is not None:
            h_out[b].wait()
        h_g = pltpu.async_copy(sh_table.at[ixv[b]], yv[b], sg[b])
        if ci + 1 < _NCH:
            h_in.wait()
            if ci + 2 < _NCH:
                h_in2 = start_in(ci + 2, b)
            compute(1 - b)
            if ci + 2 < _NCH:
                h_in = h_in2
        h_g.wait()
        h_out[b] = pltpu.async_copy(
            yv[b], out_hbm.at[pl.ds(base + ci * _CH, _CH)], so[b])
    h_out[0].wait()
    h_out[1].wait()


def kernel(coordinates, logprob_inside, baseline0, baseline1,
           W0a, b0a, W0b, W1a, b1a, W1b):
    coords = coordinates.astype(jnp.int32)
    packed = coords[:, 0] | (coords[:, 1] << 16)
    table = _build_table(
        logprob_inside.reshape(1, 1),
        baseline0.reshape(1, 8),
        baseline1,
        W0a,
        b0a.reshape(1, 10),
        W0b,
        W1a[:10],
        W1a[10:],
        b1a.reshape(1, 10),
        W1b,
    )
    return _sc_lookup(packed, table.reshape(-1))


# speedup vs baseline: 1.4754x; 1.4754x over previous
import functools
import math

import jax
import jax.numpy as jnp
import numpy as np
from jax import lax
from jax.experimental import pallas as pl
from jax.experimental.pallas import tpu as pltpu
from jax.experimental.pallas import tpu_sc as plsc

N = 1048576
WIDTH = 1024
TOTAL_WIDTH = 100000
N_FREQ = 5
NC0 = 4096
STRIDE = 128

_FREQS = np.repeat(
    1.0 / 1000.0 ** (2.0 * np.arange(1, N_FREQ + 1) / N_FREQ), 2
).astype(np.float32)
_SHIFTS = np.tile(np.array([0.0, np.pi / 2.0], dtype=np.float32), N_FREQ)

_BC = (np.arange(8, dtype=np.float32) * 128.0)[:, None]
_SVC = np.sin(_BC * _FREQS[None, :] + _SHIFTS[None, :]).astype(np.float32)

_SVC_FLAT = _SVC.reshape(1, 80)
_G = np.kron(np.eye(8, dtype=np.float32), np.ones((8, 1), np.float32))
_GT = np.ascontiguousarray(_G.T)


def _table_body(freqs_ref, shifts_ref, svcf_ref, g_ref, gt_ref,
                lpi_ref, b0_ref, b1row_ref, w0a_ref, b0a_ref, w0b_ref,
                w1apt_ref, w1abbd_ref, b1at_ref, w1bbd_ref, out_ref):
    c0 = lax.broadcasted_iota(jnp.int32, (NC0, 1), 0).astype(jnp.float32)
    emb = jnp.sin(c0 * freqs_ref[...] + shifts_ref[...])

    lpi = lpi_ref[0, 0]
    const_in = -jnp.log(1.0 + jnp.exp(-lpi)) - math.log(16.0)
    lpo = -jnp.log(1.0 + jnp.exp(lpi)) - math.log(float(TOTAL_WIDTH - WIDTH))

    dot = lambda x, y: jnp.dot(x, y, preferred_element_type=jnp.float32)

    h0 = jax.nn.sigmoid(dot(emb, w0a_ref[...]) + b0a_ref[...])
    h0 = dot(h0, w0b_ref[...]) + b0_ref[...]
    m0 = jnp.max(h0, axis=1, keepdims=True)
    ls0 = h0 - m0 - jnp.log(jnp.sum(jnp.exp(h0 - m0), axis=1, keepdims=True))

    bbin_flat = dot(svcf_ref[...], w1abbd_ref[...]) + b1at_ref[...]
    s = jax.nn.sigmoid(dot(emb, w1apt_ref[...]) + bbin_flat)
    hall = dot(s, w1bbd_ref[...]) + b1row_ref[...]
    m1 = jnp.max(hall, axis=1, keepdims=True)
    l1 = m1 + jnp.log(dot(jnp.exp(hall - m1), g_ref[...]))

    out_ref[:, 0:64] = hall + dot(ls0 - l1, gt_ref[...]) + const_in
    out_ref[:, 64:128] = jnp.broadcast_to(lpo, (NC0, 64))


def _build_table(lpi, b0, b1, w0a, b0a, w0b, w1ap, w1ab, b1a, w1b):
    eye8 = jnp.eye(8, dtype=jnp.float32)
    w1apt = jnp.concatenate([w1ap] * 8, axis=1)
    w1abbd = jnp.kron(eye8, w1ab)
    w1bbd = jnp.kron(eye8, w1b)
    b1at = jnp.concatenate([b1a] * 8, axis=1)
    return pl.pallas_call(
        _table_body,
        out_shape=jax.ShapeDtypeStruct((NC0, STRIDE), jnp.float32),
    )(jnp.asarray(_FREQS[None, :]), jnp.asarray(_SHIFTS[None, :]),
      jnp.asarray(_SVC_FLAT), jnp.asarray(_G), jnp.asarray(_GT),
      lpi, b0, b1.reshape(1, 64), w0a, b0a, w0b, w1apt, w1abbd, b1at, w1bbd)


_NCORES = 2
_NSUB = 16
_NW = _NCORES * _NSUB
_PER_W = N // _NW
_CH = 8192
_NCH = _PER_W // _CH

_sc_mesh = plsc.VectorSubcoreMesh(core_axis_name="c", subcore_axis_name="s")


@functools.partial(
    pl.kernel,
    mesh=_sc_mesh,
    out_type=jax.ShapeDtypeStruct((N,), jnp.float32),
    scratch_types=[
        pltpu.VMEM((_CH,), jnp.int32), pltpu.VMEM((_CH,), jnp.int32),
        pltpu.VMEM((_CH,), jnp.int32), pltpu.VMEM((_CH,), jnp.int32),
        pltpu.VMEM((_CH,), jnp.float32), pltpu.VMEM((_CH,), jnp.float32),
        pltpu.SemaphoreType.DMA, pltpu.SemaphoreType.DMA,
        pltpu.SemaphoreType.DMA, pltpu.SemaphoreType.DMA,
        pltpu.SemaphoreType.DMA, pltpu.SemaphoreType.DMA,
    ],
)
def _sc_lookup(coords_hbm, table_hbm, out_hbm,
               cv0, cv1, ix0, ix1, y0, y1,
               si0, si1, sg0, sg1, so0, so1):
    cv = (cv0, cv1)
    ixv = (ix0, ix1)
    yv = (y0, y1)
    sin_ = (si0, si1)
    sg = (sg0, sg1)
    so = (so0, so1)
    wid = lax.axis_index("s") * _NCORES + lax.axis_index("c")
    base = wid * _PER_W

    def start_in(ci, b):
        return pltpu.async_copy(
            coords_hbm.at[pl.ds(base + ci * _CH, _CH)], cv[b], sin_[b])

    def compute(b):
        def vec(vi, carry):
            v = cv[b][pl.ds(vi * 16, 16)]
            a = v & 0xFFFF
            fs = jnp.abs(lax.shift_right_logical(v, 16) - a)
            ixv[b][pl.ds(vi * 16, 16)] = (a << 7) + jnp.minimum(fs >> 4, 64)
            return carry
        lax.fori_loop(0, _CH // 16, vec, 0)

    h_in = start_in(0, 0)
    h_in.wait()
    compute(0)
    h_in = start_in(1, 1)
    h_out = [None, None]
    for ci in range(_NCH):
        b = ci & 1
        if h_out[b] is not None:
            h_out[b].wait()
        h_g = pltpu.async_copy(table_hbm.at[ixv[b]], yv[b], sg[b])
        if ci + 1 < _NCH:
            h_in.wait()
            if ci + 2 < _NCH:
                h_in2 = start_in(ci + 2, b)
            compute(1 - b)
            if ci + 2 < _NCH:
                h_in = h_in2
        h_g.wait()
        h_out[b] = pltpu.async_copy(
            yv[b], out_hbm.at[pl.ds(base + ci * _CH, _CH)], so[b])
    h_out[0].wait()
    h_out[1].wait()


def kernel(coordinates, logprob_inside, baseline0, baseline1,
           W0a, b0a, W0b, W1a, b1a, W1b):
    coords = coordinates.astype(jnp.int32)
    packed = coords[:, 0] | (coords[:, 1] << 16)
    table = _build_table(
        logprob_inside.reshape(1, 1),
        baseline0.reshape(1, 8),
        baseline1,
        W0a,
        b0a.reshape(1, 10),
        W0b,
        W1a[:10],
        W1a[10:],
        b1a.reshape(1, 10),
        W1b,
    )
    return _sc_lookup(packed, table.reshape(-1))
